# Initial kernel scaffold; baseline (speedup 1.0000x reference)
import jax
import jax.numpy as jnp
from jax.experimental import pallas as pl


def kernel(x, edge_index, W_l1, b_l1, W_r1, W_l2, b_l2, W_r2):
    raise NotImplementedError("write your pallas kernel here")


# speedup vs baseline: 6.6339x; 6.6339x over previous
import functools

import jax
import jax.numpy as jnp
from jax import lax
from jax.experimental import pallas as pl
from jax.experimental.pallas import tpu as pltpu
from jax.experimental.pallas import tpu_sc as plsc

N = 10000
E = 320000
D = 128
NC = 2
NS = 16
NW = NC * NS
EPW = E // NW
CH = 80
NCH = EPW // CH
NP = 10240
RPT = NP // NS
L = 16
NV = EPW // L
CR = NP // D

_DN = (((1,), (1,)), ((), ()))
_PREC = lax.Precision.HIGHEST

_mesh = plsc.VectorSubcoreMesh(core_axis_name="c", subcore_axis_name="s")


@functools.partial(
    pl.kernel,
    out_type=jax.ShapeDtypeStruct((NC, NP, D), jnp.float32),
    mesh=_mesh,
    scratch_types=[
        pltpu.VMEM((NCH, CH), jnp.int32),
        pltpu.VMEM((NCH, CH), jnp.int32),
        pltpu.VMEM((CH, D), jnp.float32),
        pltpu.VMEM_SHARED((NP, D), jnp.float32),
        pltpu.SemaphoreType.DMA,
    ],
)
def _sc_aggregate(y_hbm, src_hbm, dst_hbm, zrow_hbm, p_hbm,
                  src_v, dst_v, rows_v, acc_sh, sem):
    c = lax.axis_index("c")
    s = lax.axis_index("s")
    w = c * NS + s
    pltpu.sync_copy(src_hbm.at[w], src_v)
    pltpu.sync_copy(dst_hbm.at[w], dst_v)
    pltpu.sync_copy(zrow_hbm.at[pl.ds(s * RPT, RPT)],
                    acc_sh.at[pl.ds(s * RPT, RPT)])
    plsc.subcore_barrier()

    def chunk(j, carry):
        pltpu.async_copy(y_hbm.at[src_v.at[j]], rows_v, sem).wait()
        pltpu.sync_copy(rows_v, acc_sh.at[dst_v.at[j]], add=True)
        return carry

    lax.fori_loop(0, NCH, chunk, 0)
    plsc.subcore_barrier()
    pltpu.sync_copy(acc_sh.at[pl.ds(s * RPT, RPT)],
                    p_hbm.at[c, pl.ds(s * RPT, RPT)])


@functools.partial(
    pl.kernel,
    out_type=jax.ShapeDtypeStruct((NC, NP, D), jnp.float32),
    mesh=_mesh,
    scratch_types=[
        pltpu.VMEM((NCH, CH), jnp.int32),
        pltpu.VMEM((CH, D), jnp.float32),
        pltpu.VMEM_SHARED((NP, D), jnp.float32),
    ],
)
def _sc_count(dst_hbm, zrow_hbm, cnt_hbm, dst_v, ones_v, cnt_sh):
    c = lax.axis_index("c")
    s = lax.axis_index("s")
    w = c * NS + s
    pltpu.sync_copy(dst_hbm.at[w], dst_v)
    pltpu.sync_copy(zrow_hbm.at[pl.ds(s * RPT, RPT)],
                    cnt_sh.at[pl.ds(s * RPT, RPT)])
    one16 = jnp.full((L,), 1.0, jnp.float32)

    def fill(t, carry):
        ones_v[t // 8, pl.ds((t % 8) * L, L)] = one16
        return carry

    lax.fori_loop(0, CH * 8, fill, 0)
    plsc.subcore_barrier()

    def chunk(j, carry):
        pltpu.sync_copy(ones_v, cnt_sh.at[dst_v.at[j]], add=True)
        return carry

    lax.fori_loop(0, NCH, chunk, 0)
    plsc.subcore_barrier()
    pltpu.sync_copy(cnt_sh.at[pl.ds(s * RPT, RPT)],
                    cnt_hbm.at[c, pl.ds(s * RPT, RPT)])


RB = 1024
GB = RB // D


def _lin_body(x_ref, wl_ref, wr_ref, b_ref, y_ref, z_ref):
    xv = x_ref[...]
    y_ref[...] = lax.dot_general(xv, wl_ref[...], _DN, precision=_PREC)
    z_ref[...] = lax.dot_general(xv, wr_ref[...], _DN, precision=_PREC) + b_ref[...]


def _lin(x, wl, wr, b):
    return pl.pallas_call(
        _lin_body,
        grid=(NP // RB,),
        in_specs=[pl.BlockSpec((RB, D), lambda i: (i, 0)),
                  pl.BlockSpec((D, D), lambda i: (0, 0)),
                  pl.BlockSpec((D, D), lambda i: (0, 0)),
                  pl.BlockSpec((1, D), lambda i: (0, 0))],
        out_specs=[pl.BlockSpec((RB, D), lambda i: (i, 0)),
                   pl.BlockSpec((RB, D), lambda i: (i, 0))],
        out_shape=[jax.ShapeDtypeStruct((NP, D), jnp.float32)] * 2,
    )(x, wl, wr, b.reshape(1, D))


def _mean(p_ref, cnt_ref):
    cnt = cnt_ref[0] + cnt_ref[1]
    return (p_ref[0] + p_ref[1]) / jnp.maximum(cnt, 1.0)


def _comb_body(p_ref, cnt_ref, z_ref, wl_ref, wr_ref, b_ref, y_ref, z2_ref):
    h = jnp.maximum(_mean(p_ref, cnt_ref) + z_ref[...], 0.0)
    y_ref[...] = lax.dot_general(h, wl_ref[...], _DN, precision=_PREC)
    z2_ref[...] = lax.dot_general(h, wr_ref[...], _DN, precision=_PREC) + b_ref[...]


def _comb(p, cnt, z, wl, wr, b):
    return pl.pallas_call(
        _comb_body,
        grid=(NP // RB,),
        in_specs=[pl.BlockSpec((NC, RB, D), lambda i: (0, i, 0)),
                  pl.BlockSpec((NC, RB, D), lambda i: (0, i, 0)),
                  pl.BlockSpec((RB, D), lambda i: (i, 0)),
                  pl.BlockSpec((D, D), lambda i: (0, 0)),
                  pl.BlockSpec((D, D), lambda i: (0, 0)),
                  pl.BlockSpec((1, D), lambda i: (0, 0))],
        out_specs=[pl.BlockSpec((RB, D), lambda i: (i, 0)),
                   pl.BlockSpec((RB, D), lambda i: (i, 0))],
        out_shape=[jax.ShapeDtypeStruct((NP, D), jnp.float32)] * 2,
    )(p, cnt, z, wl, wr, b.reshape(1, D))


def _final_body(p_ref, cnt_ref, z_ref, o_ref):
    o_ref[...] = _mean(p_ref, cnt_ref) + z_ref[...]


def _final(p, cnt, z):
    return pl.pallas_call(
        _final_body,
        grid=(NP // RB,),
        in_specs=[pl.BlockSpec((NC, RB, D), lambda i: (0, i, 0)),
                  pl.BlockSpec((NC, RB, D), lambda i: (0, i, 0)),
                  pl.BlockSpec((RB, D), lambda i: (i, 0))],
        out_specs=pl.BlockSpec((RB, D), lambda i: (i, 0)),
        out_shape=jax.ShapeDtypeStruct((NP, D), jnp.float32),
    )(p, cnt, z)


def kernel(x, edge_index, W_l1, b_l1, W_r1, W_l2, b_l2, W_r2):
    ei = edge_index.astype(jnp.int32)
    src = ei[0].reshape(NW, NCH, CH)
    dst = ei[1].reshape(NW, NCH, CH)
    zrow = jnp.zeros((NP, D), jnp.float32)
    x_pad = jnp.concatenate([x, jnp.zeros((NP - N, D), jnp.float32)], axis=0)

    cnt = _sc_count(dst, zrow)
    y1, z1 = _lin(x_pad, W_l1, W_r1, b_l1)
    p1 = _sc_aggregate(y1, src, dst, zrow)
    y2, z2 = _comb(p1, cnt, z1, W_l2, W_r2, b_l2)
    p2 = _sc_aggregate(y2, src, dst, zrow)
    return _final(p2, cnt, z2)[:N]


# speedup vs baseline: 9.5862x; 1.4450x over previous
import functools

import jax
import jax.numpy as jnp
from jax import lax
from jax.experimental import pallas as pl
from jax.experimental.pallas import tpu as pltpu
from jax.experimental.pallas import tpu_sc as plsc

N = 10000
E = 320000
D = 128
NC = 2
NS = 16
NW = NC * NS
EPW = E // NW
CH = 80
NCH = EPW // CH
NP = 10240
RPT = NP // NS
L = 16
NV = EPW // L
CR = NP // D

_DN = (((1,), (1,)), ((), ()))
_PREC = lax.Precision.HIGHEST

_mesh = plsc.VectorSubcoreMesh(core_axis_name="c", subcore_axis_name="s")


@functools.partial(
    pl.kernel,
    out_type=jax.ShapeDtypeStruct((NC, NP, D), jnp.float32),
    mesh=_mesh,
    scratch_types=[
        pltpu.VMEM((EPW,), jnp.int32),
        pltpu.VMEM((NCH, CH), jnp.int32),
        pltpu.VMEM((2, CH, D), jnp.float32),
        pltpu.VMEM_SHARED((NP, D), jnp.float32),
        pltpu.SemaphoreType.DMA,
        pltpu.SemaphoreType.DMA,
    ],
)
def _sc_aggregate(y_hbm, srcf_hbm, dst_hbm, zrow_hbm, p_hbm,
                  src_v, dst_v, rows_v, acc_sh, g0, g1):
    c = lax.axis_index("c")
    s = lax.axis_index("s")
    w = c * NS + s
    gsems = (g0, g1)
    pltpu.sync_copy(srcf_hbm.at[w], src_v)
    pltpu.sync_copy(dst_hbm.at[w], dst_v)
    pltpu.sync_copy(zrow_hbm.at[pl.ds(s * RPT, RPT)],
                    acc_sh.at[pl.ds(s * RPT, RPT)])
    plsc.subcore_barrier()

    def gather(j, b):
        return pltpu.async_copy(y_hbm.at[src_v.at[pl.ds(j * CH, CH)]],
                                rows_v.at[b], gsems[b])

    gather(0, 0)
    gather(1, 1)

    def pair(t, carry):
        for b in (0, 1):
            j = 2 * t + b
            pltpu.make_async_copy(y_hbm.at[src_v.at[pl.ds(j * CH, CH)]],
                                  rows_v.at[b], gsems[b]).wait()
            pltpu.sync_copy(rows_v.at[b], acc_sh.at[dst_v.at[j]], add=True)
            jn = j + 2

            @pl.when(jn < NCH)
            def _():
                gather(jn, b)
        return carry

    lax.fori_loop(0, NCH // 2, pair, 0)
    if NCH % 2:
        j = NCH - 1
        pltpu.make_async_copy(y_hbm.at[src_v.at[pl.ds(j * CH, CH)]],
                              rows_v.at[0], g0).wait()
        pltpu.sync_copy(rows_v.at[0], acc_sh.at[dst_v.at[j]], add=True)
    plsc.subcore_barrier()
    pltpu.sync_copy(acc_sh.at[pl.ds(s * RPT, RPT)],
                    p_hbm.at[c, pl.ds(s * RPT, RPT)])


@functools.partial(
    pl.kernel,
    out_type=jax.ShapeDtypeStruct((NC, NP, D), jnp.float32),
    mesh=_mesh,
    scratch_types=[
        pltpu.VMEM((NCH, CH), jnp.int32),
        pltpu.VMEM((CH, D), jnp.float32),
        pltpu.VMEM_SHARED((NP, D), jnp.float32),
        pltpu.SemaphoreType.DMA,
        pltpu.SemaphoreType.DMA,
        pltpu.SemaphoreType.DMA,
        pltpu.SemaphoreType.DMA,
    ],
)
def _sc_count(dst_hbm, zrow_hbm, cnt_hbm, dst_v, ones_v, cnt_sh,
              s0, s1, s2, s3):
    ssems = (s0, s1, s2, s3)
    c = lax.axis_index("c")
    s = lax.axis_index("s")
    w = c * NS + s
    pltpu.sync_copy(dst_hbm.at[w], dst_v)
    pltpu.sync_copy(zrow_hbm.at[pl.ds(s * RPT, RPT)],
                    cnt_sh.at[pl.ds(s * RPT, RPT)])
    one16 = jnp.full((L,), 1.0, jnp.float32)

    def fill(t, carry):
        ones_v[t // 8, pl.ds((t % 8) * L, L)] = one16
        return carry

    lax.fori_loop(0, CH * 8, fill, 0)
    plsc.subcore_barrier()

    def batch(t, carry):
        j0 = t * 4
        scp = [pltpu.async_copy(ones_v, cnt_sh.at[dst_v.at[j0 + b]],
                                ssems[b], add=True) for b in range(4)]
        for b in range(4):
            scp[b].wait()
        return carry

    lax.fori_loop(0, NCH // 4, batch, 0)
    for j in range(NCH - NCH % 4, NCH):
        pltpu.sync_copy(ones_v, cnt_sh.at[dst_v.at[j]], add=True)
    plsc.subcore_barrier()
    pltpu.sync_copy(cnt_sh.at[pl.ds(s * RPT, RPT)],
                    cnt_hbm.at[c, pl.ds(s * RPT, RPT)])


RB = 1024
GB = RB // D


def _lin_body(x_ref, wl_ref, wr_ref, b_ref, y_ref, z_ref):
    xv = x_ref[...]
    y_ref[...] = lax.dot_general(xv, wl_ref[...], _DN, precision=_PREC)
    z_ref[...] = lax.dot_general(xv, wr_ref[...], _DN, precision=_PREC) + b_ref[...]


def _lin(x, wl, wr, b):
    return pl.pallas_call(
        _lin_body,
        grid=(NP // RB,),
        in_specs=[pl.BlockSpec((RB, D), lambda i: (i, 0)),
                  pl.BlockSpec((D, D), lambda i: (0, 0)),
                  pl.BlockSpec((D, D), lambda i: (0, 0)),
                  pl.BlockSpec((1, D), lambda i: (0, 0))],
        out_specs=[pl.BlockSpec((RB, D), lambda i: (i, 0)),
                   pl.BlockSpec((RB, D), lambda i: (i, 0))],
        out_shape=[jax.ShapeDtypeStruct((NP, D), jnp.float32)] * 2,
    )(x, wl, wr, b.reshape(1, D))


def _mean(p_ref, cnt_ref):
    cnt = cnt_ref[0] + cnt_ref[1]
    return (p_ref[0] + p_ref[1]) / jnp.maximum(cnt, 1.0)


def _comb_body(p_ref, cnt_ref, z_ref, wl_ref, wr_ref, b_ref, y_ref, z2_ref):
    h = jnp.maximum(_mean(p_ref, cnt_ref) + z_ref[...], 0.0)
    y_ref[...] = lax.dot_general(h, wl_ref[...], _DN, precision=_PREC)
    z2_ref[...] = lax.dot_general(h, wr_ref[...], _DN, precision=_PREC) + b_ref[...]


def _comb(p, cnt, z, wl, wr, b):
    return pl.pallas_call(
        _comb_body,
        grid=(NP // RB,),
        in_specs=[pl.BlockSpec((NC, RB, D), lambda i: (0, i, 0)),
                  pl.BlockSpec((NC, RB, D), lambda i: (0, i, 0)),
                  pl.BlockSpec((RB, D), lambda i: (i, 0)),
                  pl.BlockSpec((D, D), lambda i: (0, 0)),
                  pl.BlockSpec((D, D), lambda i: (0, 0)),
                  pl.BlockSpec((1, D), lambda i: (0, 0))],
        out_specs=[pl.BlockSpec((RB, D), lambda i: (i, 0)),
                   pl.BlockSpec((RB, D), lambda i: (i, 0))],
        out_shape=[jax.ShapeDtypeStruct((NP, D), jnp.float32)] * 2,
    )(p, cnt, z, wl, wr, b.reshape(1, D))


def _final_body(p_ref, cnt_ref, z_ref, o_ref):
    o_ref[...] = _mean(p_ref, cnt_ref) + z_ref[...]


def _final(p, cnt, z):
    return pl.pallas_call(
        _final_body,
        grid=(NP // RB,),
        in_specs=[pl.BlockSpec((NC, RB, D), lambda i: (0, i, 0)),
                  pl.BlockSpec((NC, RB, D), lambda i: (0, i, 0)),
                  pl.BlockSpec((RB, D), lambda i: (i, 0))],
        out_specs=pl.BlockSpec((RB, D), lambda i: (i, 0)),
        out_shape=jax.ShapeDtypeStruct((NP, D), jnp.float32),
    )(p, cnt, z)


def kernel(x, edge_index, W_l1, b_l1, W_r1, W_l2, b_l2, W_r2):
    ei = edge_index.astype(jnp.int32)
    srcf = ei[0].reshape(NW, EPW)
    dst = ei[1].reshape(NW, NCH, CH)
    zrow = jnp.zeros((NP, D), jnp.float32)
    x_pad = jnp.concatenate([x, jnp.zeros((NP - N, D), jnp.float32)], axis=0)

    cnt = _sc_count(dst, zrow)
    y1, z1 = _lin(x_pad, W_l1, W_r1, b_l1)
    p1 = _sc_aggregate(y1, srcf, dst, zrow)
    y2, z2 = _comb(p1, cnt, z1, W_l2, W_r2, b_l2)
    p2 = _sc_aggregate(y2, srcf, dst, zrow)
    return _final(p2, cnt, z2)[:N]


# speedup vs baseline: 9.6218x; 1.0037x over previous
import functools

import jax
import jax.numpy as jnp
from jax import lax
from jax.experimental import pallas as pl
from jax.experimental.pallas import tpu as pltpu
from jax.experimental.pallas import tpu_sc as plsc

N = 10000
E = 320000
D = 128
NC = 2
NS = 16
NW = NC * NS
EPW = E // NW
CH = 40
NCH = EPW // CH
NP = 10240
RPT = NP // NS
L = 16
NV = EPW // L
CR = NP // D

_DN = (((1,), (1,)), ((), ()))
_PREC = lax.Precision.HIGHEST

_mesh = plsc.VectorSubcoreMesh(core_axis_name="c", subcore_axis_name="s")


@functools.partial(
    pl.kernel,
    out_type=jax.ShapeDtypeStruct((NC, NP, D), jnp.float32),
    mesh=_mesh,
    scratch_types=[
        pltpu.VMEM((EPW,), jnp.int32),
        pltpu.VMEM((EPW,), jnp.int32),
        pltpu.VMEM((4, CH, D), jnp.float32),
        pltpu.VMEM_SHARED((NP, D), jnp.float32),
        pltpu.SemaphoreType.DMA,
        pltpu.SemaphoreType.DMA,
        pltpu.SemaphoreType.DMA,
        pltpu.SemaphoreType.DMA,
        pltpu.SemaphoreType.DMA,
        pltpu.SemaphoreType.DMA,
        pltpu.SemaphoreType.DMA,
        pltpu.SemaphoreType.DMA,
    ],
)
def _sc_aggregate(y_hbm, srcf_hbm, dstf_hbm, zrow_hbm, p_hbm,
                  src_v, dst_v, rows_v, acc_sh,
                  g0, g1, g2, g3, s0, s1, s2, s3):
    c = lax.axis_index("c")
    s = lax.axis_index("s")
    w = c * NS + s
    gsems = (g0, g1, g2, g3)
    ssems = (s0, s1, s2, s3)
    pltpu.sync_copy(srcf_hbm.at[w], src_v)
    pltpu.sync_copy(dstf_hbm.at[w], dst_v)
    pltpu.sync_copy(zrow_hbm.at[pl.ds(s * RPT, RPT)],
                    acc_sh.at[pl.ds(s * RPT, RPT)])
    plsc.subcore_barrier()

    def gather_desc(j, b):
        return pltpu.make_async_copy(y_hbm.at[src_v.at[pl.ds(j * CH, CH)]],
                                     rows_v.at[b], gsems[b])

    def scatter_desc(j, b):
        return pltpu.make_async_copy(rows_v.at[b],
                                     acc_sh.at[dst_v.at[pl.ds(j * CH, CH)]],
                                     ssems[b])

    def gather(j, b):
        gather_desc(j, b).start()

    def scatter(j, b):
        scatter_desc(j, b).start(add=True)

    for b in range(4):
        gather(b, b)

    def superstep(t, carry):
        j0 = 4 * t
        for b in range(4):
            gather_desc(j0 + b, b).wait()
            scatter(j0 + b, b)
        for b in range(4):
            jn = j0 + b + 4
            scatter_desc(j0 + b, b).wait()

            @pl.when(jn < NCH)
            def _():
                gather(jn, b)
        return carry

    lax.fori_loop(0, NCH // 4, superstep, 0)
    for j in range(NCH - NCH % 4, NCH):
        b = j % 4
        gather_desc(j, b).wait()
        pltpu.sync_copy(rows_v.at[b], acc_sh.at[dst_v.at[pl.ds(j * CH, CH)]],
                        add=True)
    plsc.subcore_barrier()
    pltpu.sync_copy(acc_sh.at[pl.ds(s * RPT, RPT)],
                    p_hbm.at[c, pl.ds(s * RPT, RPT)])


@functools.partial(
    pl.kernel,
    out_type=jax.ShapeDtypeStruct((NC, NP, D), jnp.float32),
    mesh=_mesh,
    scratch_types=[
        pltpu.VMEM((EPW,), jnp.int32),
        pltpu.VMEM((CH, D), jnp.float32),
        pltpu.VMEM_SHARED((NP, D), jnp.float32),
        pltpu.SemaphoreType.DMA,
        pltpu.SemaphoreType.DMA,
        pltpu.SemaphoreType.DMA,
        pltpu.SemaphoreType.DMA,
    ],
)
def _sc_count(dstf_hbm, zrow_hbm, cnt_hbm, dst_v, ones_v, cnt_sh,
              s0, s1, s2, s3):
    ssems = (s0, s1, s2, s3)
    c = lax.axis_index("c")
    s = lax.axis_index("s")
    w = c * NS + s
    pltpu.sync_copy(dstf_hbm.at[w], dst_v)
    pltpu.sync_copy(zrow_hbm.at[pl.ds(s * RPT, RPT)],
                    cnt_sh.at[pl.ds(s * RPT, RPT)])
    one16 = jnp.full((L,), 1.0, jnp.float32)

    def fill(t, carry):
        ones_v[t // 8, pl.ds((t % 8) * L, L)] = one16
        return carry

    lax.fori_loop(0, CH * 8, fill, 0)
    plsc.subcore_barrier()

    def scatter_desc(j, b):
        return pltpu.make_async_copy(ones_v,
                                     cnt_sh.at[dst_v.at[pl.ds(j * CH, CH)]],
                                     ssems[b])

    def batch(t, carry):
        j0 = t * 4
        for b in range(4):
            scatter_desc(j0 + b, b).start(add=True)
        for b in range(4):
            scatter_desc(j0 + b, b).wait()
        return carry

    lax.fori_loop(0, NCH // 4, batch, 0)
    for j in range(NCH - NCH % 4, NCH):
        pltpu.sync_copy(ones_v, cnt_sh.at[dst_v.at[pl.ds(j * CH, CH)]],
                        add=True)
    plsc.subcore_barrier()
    pltpu.sync_copy(cnt_sh.at[pl.ds(s * RPT, RPT)],
                    cnt_hbm.at[c, pl.ds(s * RPT, RPT)])


RB = 1024
GB = RB // D


def _lin_body(x_ref, wl_ref, wr_ref, b_ref, y_ref, z_ref):
    xv = x_ref[...]
    y_ref[...] = lax.dot_general(xv, wl_ref[...], _DN, precision=_PREC)
    z_ref[...] = lax.dot_general(xv, wr_ref[...], _DN, precision=_PREC) + b_ref[...]


def _lin(x, wl, wr, b):
    return pl.pallas_call(
        _lin_body,
        grid=(NP // RB,),
        in_specs=[pl.BlockSpec((RB, D), lambda i: (i, 0)),
                  pl.BlockSpec((D, D), lambda i: (0, 0)),
                  pl.BlockSpec((D, D), lambda i: (0, 0)),
                  pl.BlockSpec((1, D), lambda i: (0, 0))],
        out_specs=[pl.BlockSpec((RB, D), lambda i: (i, 0)),
                   pl.BlockSpec((RB, D), lambda i: (i, 0))],
        out_shape=[jax.ShapeDtypeStruct((NP, D), jnp.float32)] * 2,
    )(x, wl, wr, b.reshape(1, D))


def _mean(p_ref, cnt_ref):
    cnt = cnt_ref[0] + cnt_ref[1]
    return (p_ref[0] + p_ref[1]) / jnp.maximum(cnt, 1.0)


def _comb_body(p_ref, cnt_ref, z_ref, wl_ref, wr_ref, b_ref, y_ref, z2_ref):
    h = jnp.maximum(_mean(p_ref, cnt_ref) + z_ref[...], 0.0)
    y_ref[...] = lax.dot_general(h, wl_ref[...], _DN, precision=_PREC)
    z2_ref[...] = lax.dot_general(h, wr_ref[...], _DN, precision=_PREC) + b_ref[...]


def _comb(p, cnt, z, wl, wr, b):
    return pl.pallas_call(
        _comb_body,
        grid=(NP // RB,),
        in_specs=[pl.BlockSpec((NC, RB, D), lambda i: (0, i, 0)),
                  pl.BlockSpec((NC, RB, D), lambda i: (0, i, 0)),
                  pl.BlockSpec((RB, D), lambda i: (i, 0)),
                  pl.BlockSpec((D, D), lambda i: (0, 0)),
                  pl.BlockSpec((D, D), lambda i: (0, 0)),
                  pl.BlockSpec((1, D), lambda i: (0, 0))],
        out_specs=[pl.BlockSpec((RB, D), lambda i: (i, 0)),
                   pl.BlockSpec((RB, D), lambda i: (i, 0))],
        out_shape=[jax.ShapeDtypeStruct((NP, D), jnp.float32)] * 2,
    )(p, cnt, z, wl, wr, b.reshape(1, D))


def _final_body(p_ref, cnt_ref, z_ref, o_ref):
    o_ref[...] = _mean(p_ref, cnt_ref) + z_ref[...]


def _final(p, cnt, z):
    return pl.pallas_call(
        _final_body,
        grid=(NP // RB,),
        in_specs=[pl.BlockSpec((NC, RB, D), lambda i: (0, i, 0)),
                  pl.BlockSpec((NC, RB, D), lambda i: (0, i, 0)),
                  pl.BlockSpec((RB, D), lambda i: (i, 0))],
        out_specs=pl.BlockSpec((RB, D), lambda i: (i, 0)),
        out_shape=jax.ShapeDtypeStruct((NP, D), jnp.float32),
    )(p, cnt, z)


def kernel(x, edge_index, W_l1, b_l1, W_r1, W_l2, b_l2, W_r2):
    ei = edge_index.astype(jnp.int32)
    srcf = ei[0].reshape(NW, EPW)
    dstf = ei[1].reshape(NW, EPW)
    zrow = jnp.zeros((NP, D), jnp.float32)
    x_pad = jnp.concatenate([x, jnp.zeros((NP - N, D), jnp.float32)], axis=0)

    cnt = _sc_count(dstf, zrow)
    y1, z1 = _lin(x_pad, W_l1, W_r1, b_l1)
    p1 = _sc_aggregate(y1, srcf, dstf, zrow)
    y2, z2 = _comb(p1, cnt, z1, W_l2, W_r2, b_l2)
    p2 = _sc_aggregate(y2, srcf, dstf, zrow)
    return _final(p2, cnt, z2)[:N]


# speedup vs baseline: 9.8447x; 1.0232x over previous
import functools

import jax
import jax.numpy as jnp
from jax import lax
from jax.experimental import pallas as pl
from jax.experimental.pallas import tpu as pltpu
from jax.experimental.pallas import tpu_sc as plsc

N = 10000
E = 320000
D = 128
NC = 2
NS = 16
NW = NC * NS
EPW = E // NW
CH = 40
NCH = EPW // CH
NP = 10240
RPT = NP // NS
L = 16
NV = EPW // L
CR = NP // D

_DN = (((1,), (1,)), ((), ()))
_PREC = lax.Precision.HIGHEST

_mesh = plsc.VectorSubcoreMesh(core_axis_name="c", subcore_axis_name="s")


@functools.partial(
    pl.kernel,
    out_type=jax.ShapeDtypeStruct((NC, NP, D), jnp.float32),
    mesh=_mesh,
    scratch_types=[
        pltpu.VMEM((EPW,), jnp.int32),
        pltpu.VMEM((EPW,), jnp.int32),
        pltpu.VMEM((4, CH, D), jnp.float32),
        pltpu.VMEM_SHARED((NP, D), jnp.float32),
        pltpu.SemaphoreType.DMA,
        pltpu.SemaphoreType.DMA,
        pltpu.SemaphoreType.DMA,
        pltpu.SemaphoreType.DMA,
        pltpu.SemaphoreType.DMA,
        pltpu.SemaphoreType.DMA,
        pltpu.SemaphoreType.DMA,
        pltpu.SemaphoreType.DMA,
    ],
)
def _sc_aggregate(y_hbm, srcf_hbm, dstf_hbm, zrow_hbm, p_hbm,
                  src_v, dst_v, rows_v, acc_sh,
                  g0, g1, g2, g3, s0, s1, s2, s3):
    c = lax.axis_index("c")
    s = lax.axis_index("s")
    w = c * NS + s
    gsems = (g0, g1, g2, g3)
    ssems = (s0, s1, s2, s3)
    pltpu.sync_copy(srcf_hbm.at[w], src_v)
    pltpu.sync_copy(dstf_hbm.at[w], dst_v)
    pltpu.sync_copy(zrow_hbm.at[pl.ds(s * RPT, RPT)],
                    acc_sh.at[pl.ds(s * RPT, RPT)])
    plsc.subcore_barrier()

    def gather_desc(j, b):
        return pltpu.make_async_copy(y_hbm.at[src_v.at[pl.ds(j * CH, CH)]],
                                     rows_v.at[b], gsems[b])

    def scatter_desc(j, b):
        return pltpu.make_async_copy(rows_v.at[b],
                                     acc_sh.at[dst_v.at[pl.ds(j * CH, CH)]],
                                     ssems[b])

    def gather(j, b):
        gather_desc(j, b).start()

    def scatter(j, b):
        scatter_desc(j, b).start(add=True)

    for b in range(4):
        gather(b, b)

    def superstep(t, carry):
        j0 = 4 * t
        for b in range(4):
            gather_desc(j0 + b, b).wait()
            scatter(j0 + b, b)
        for b in range(4):
            jn = j0 + b + 4
            scatter_desc(j0 + b, b).wait()

            @pl.when(jn < NCH)
            def _():
                gather(jn, b)
        return carry

    lax.fori_loop(0, NCH // 4, superstep, 0)
    for j in range(NCH - NCH % 4, NCH):
        b = j % 4
        gather_desc(j, b).wait()
        pltpu.sync_copy(rows_v.at[b], acc_sh.at[dst_v.at[pl.ds(j * CH, CH)]],
                        add=True)
    plsc.subcore_barrier()
    pltpu.sync_copy(acc_sh.at[pl.ds(s * RPT, RPT)],
                    p_hbm.at[c, pl.ds(s * RPT, RPT)])


@functools.partial(
    pl.kernel,
    out_type=jax.ShapeDtypeStruct((NC, NP, D), jnp.float32),
    mesh=_mesh,
    scratch_types=[
        pltpu.VMEM((EPW,), jnp.int32),
        pltpu.VMEM((CH, D), jnp.float32),
        pltpu.VMEM_SHARED((NP, D), jnp.float32),
        pltpu.SemaphoreType.DMA,
        pltpu.SemaphoreType.DMA,
        pltpu.SemaphoreType.DMA,
        pltpu.SemaphoreType.DMA,
    ],
)
def _sc_count(dstf_hbm, zrow_hbm, cnt_hbm, dst_v, ones_v, cnt_sh,
              s0, s1, s2, s3):
    ssems = (s0, s1, s2, s3)
    c = lax.axis_index("c")
    s = lax.axis_index("s")
    w = c * NS + s
    pltpu.sync_copy(dstf_hbm.at[w], dst_v)
    pltpu.sync_copy(zrow_hbm.at[pl.ds(s * RPT, RPT)],
                    cnt_sh.at[pl.ds(s * RPT, RPT)])
    one16 = jnp.full((L,), 1.0, jnp.float32)

    def fill(t, carry):
        ones_v[t // 8, pl.ds((t % 8) * L, L)] = one16
        return carry

    lax.fori_loop(0, CH * 8, fill, 0)
    plsc.subcore_barrier()

    def scatter_desc(j, b):
        return pltpu.make_async_copy(ones_v,
                                     cnt_sh.at[dst_v.at[pl.ds(j * CH, CH)]],
                                     ssems[b])

    def batch(t, carry):
        j0 = t * 4
        for b in range(4):
            scatter_desc(j0 + b, b).start(add=True)
        for b in range(4):
            scatter_desc(j0 + b, b).wait()
        return carry

    lax.fori_loop(0, NCH // 4, batch, 0)
    for j in range(NCH - NCH % 4, NCH):
        pltpu.sync_copy(ones_v, cnt_sh.at[dst_v.at[pl.ds(j * CH, CH)]],
                        add=True)
    plsc.subcore_barrier()
    pltpu.sync_copy(cnt_sh.at[pl.ds(s * RPT, RPT)],
                    cnt_hbm.at[c, pl.ds(s * RPT, RPT)])


RB = 1024
GB = RB // D


def _lin_body(x_ref, wl_ref, wr_ref, b_ref, y_ref, z_ref):
    xv = x_ref[...]
    y_ref[...] = lax.dot_general(xv, wl_ref[...], _DN, precision=_PREC)
    z_ref[...] = lax.dot_general(xv, wr_ref[...], _DN, precision=_PREC) + b_ref[...]


def _lin(x, wl, wr, b):
    return pl.pallas_call(
        _lin_body,
        grid=(pl.cdiv(N, RB),),
        in_specs=[pl.BlockSpec((RB, D), lambda i: (i, 0)),
                  pl.BlockSpec((D, D), lambda i: (0, 0)),
                  pl.BlockSpec((D, D), lambda i: (0, 0)),
                  pl.BlockSpec((1, D), lambda i: (0, 0))],
        out_specs=[pl.BlockSpec((RB, D), lambda i: (i, 0)),
                   pl.BlockSpec((RB, D), lambda i: (i, 0))],
        out_shape=[jax.ShapeDtypeStruct((N, D), jnp.float32)] * 2,
    )(x, wl, wr, b.reshape(1, D))


def _mean(p_ref, cnt_ref):
    cnt = cnt_ref[0] + cnt_ref[1]
    return (p_ref[0] + p_ref[1]) / jnp.maximum(cnt, 1.0)


def _comb_body(p_ref, cnt_ref, z_ref, wl_ref, wr_ref, b_ref, y_ref, z2_ref):
    h = jnp.maximum(_mean(p_ref, cnt_ref) + z_ref[...], 0.0)
    y_ref[...] = lax.dot_general(h, wl_ref[...], _DN, precision=_PREC)
    z2_ref[...] = lax.dot_general(h, wr_ref[...], _DN, precision=_PREC) + b_ref[...]


def _comb(p, cnt, z, wl, wr, b):
    return pl.pallas_call(
        _comb_body,
        grid=(pl.cdiv(N, RB),),
        in_specs=[pl.BlockSpec((NC, RB, D), lambda i: (0, i, 0)),
                  pl.BlockSpec((NC, RB, D), lambda i: (0, i, 0)),
                  pl.BlockSpec((RB, D), lambda i: (i, 0)),
                  pl.BlockSpec((D, D), lambda i: (0, 0)),
                  pl.BlockSpec((D, D), lambda i: (0, 0)),
                  pl.BlockSpec((1, D), lambda i: (0, 0))],
        out_specs=[pl.BlockSpec((RB, D), lambda i: (i, 0)),
                   pl.BlockSpec((RB, D), lambda i: (i, 0))],
        out_shape=[jax.ShapeDtypeStruct((N, D), jnp.float32)] * 2,
    )(p, cnt, z, wl, wr, b.reshape(1, D))


def _final_body(p_ref, cnt_ref, z_ref, o_ref):
    o_ref[...] = _mean(p_ref, cnt_ref) + z_ref[...]


def _final(p, cnt, z):
    return pl.pallas_call(
        _final_body,
        grid=(pl.cdiv(N, RB),),
        in_specs=[pl.BlockSpec((NC, RB, D), lambda i: (0, i, 0)),
                  pl.BlockSpec((NC, RB, D), lambda i: (0, i, 0)),
                  pl.BlockSpec((RB, D), lambda i: (i, 0))],
        out_specs=pl.BlockSpec((RB, D), lambda i: (i, 0)),
        out_shape=jax.ShapeDtypeStruct((N, D), jnp.float32),
    )(p, cnt, z)


def kernel(x, edge_index, W_l1, b_l1, W_r1, W_l2, b_l2, W_r2):
    ei = edge_index.astype(jnp.int32)
    srcf = ei[0].reshape(NW, EPW)
    dstf = ei[1].reshape(NW, EPW)
    zrow = jnp.zeros((NP, D), jnp.float32)

    cnt = _sc_count(dstf, zrow)
    y1, z1 = _lin(x, W_l1, W_r1, b_l1)
    p1 = _sc_aggregate(y1, srcf, dstf, zrow)
    y2, z2 = _comb(p1, cnt, z1, W_l2, W_r2, b_l2)
    p2 = _sc_aggregate(y2, srcf, dstf, zrow)
    return _final(p2, cnt, z2)


# speedup vs baseline: 10.1783x; 1.0339x over previous
import functools

import jax
import jax.numpy as jnp
from jax import lax
from jax.experimental import pallas as pl
from jax.experimental.pallas import tpu as pltpu
from jax.experimental.pallas import tpu_sc as plsc

N = 10000
E = 320000
D = 128
NC = 2
NS = 16
NW = NC * NS
EPW = E // NW
CH = 40
NCH = EPW // CH
NP = 10240
RPT = NP // NS
L = 16
NV = EPW // L
CR = NP // D

_DN = (((1,), (1,)), ((), ()))
_PREC = lax.Precision.HIGHEST

_mesh = plsc.VectorSubcoreMesh(core_axis_name="c", subcore_axis_name="s")


@functools.partial(
    pl.kernel,
    out_type=jax.ShapeDtypeStruct((NC, NP, D), jnp.float32),
    mesh=_mesh,
    scratch_types=[
        pltpu.VMEM((EPW,), jnp.int32),
        pltpu.VMEM((EPW,), jnp.int32),
        pltpu.VMEM((5, CH, D), jnp.float32),
        pltpu.VMEM_SHARED((NP, D), jnp.float32),
        pltpu.SemaphoreType.DMA,
        pltpu.SemaphoreType.DMA,
        pltpu.SemaphoreType.DMA,
        pltpu.SemaphoreType.DMA,
        pltpu.SemaphoreType.DMA,
        pltpu.SemaphoreType.DMA,
        pltpu.SemaphoreType.DMA,
        pltpu.SemaphoreType.DMA,
        pltpu.SemaphoreType.DMA,
        pltpu.SemaphoreType.DMA,
    ],
)
def _sc_aggregate(y_hbm, srcf_hbm, dstf_hbm, zrow_hbm, p_hbm,
                  src_v, dst_v, rows_v, acc_sh,
                  g0, g1, g2, g3, g4, s0, s1, s2, s3, s4):
    c = lax.axis_index("c")
    s = lax.axis_index("s")
    w = c * NS + s
    gsems = (g0, g1, g2, g3, g4)
    ssems = (s0, s1, s2, s3, s4)
    pltpu.sync_copy(srcf_hbm.at[w], src_v)
    pltpu.sync_copy(dstf_hbm.at[w], dst_v)

    def gather_desc(j, b):
        return pltpu.make_async_copy(y_hbm.at[src_v.at[pl.ds(j * CH, CH)]],
                                     rows_v.at[b], gsems[b])

    def scatter_desc(j, b):
        return pltpu.make_async_copy(rows_v.at[b],
                                     acc_sh.at[dst_v.at[pl.ds(j * CH, CH)]],
                                     ssems[b])

    def gather(j, b):
        gather_desc(j, b).start()

    def scatter(j, b):
        scatter_desc(j, b).start(add=True)

    for b in range(5):
        gather(b, b)
    pltpu.sync_copy(zrow_hbm.at[pl.ds(s * RPT, RPT)],
                    acc_sh.at[pl.ds(s * RPT, RPT)])
    plsc.subcore_barrier()

    def superstep(t, carry):
        j0 = 5 * t
        for b in range(5):
            gather_desc(j0 + b, b).wait()
            scatter(j0 + b, b)
        for b in range(5):
            jn = j0 + b + 5
            scatter_desc(j0 + b, b).wait()

            @pl.when(jn < NCH)
            def _():
                gather(jn, b)
        return carry

    lax.fori_loop(0, NCH // 5, superstep, 0)
    plsc.subcore_barrier()
    pltpu.sync_copy(acc_sh.at[pl.ds(s * RPT, RPT)],
                    p_hbm.at[c, pl.ds(s * RPT, RPT)])


@functools.partial(
    pl.kernel,
    out_type=jax.ShapeDtypeStruct((NC, NP, D), jnp.float32),
    mesh=_mesh,
    scratch_types=[
        pltpu.VMEM((EPW,), jnp.int32),
        pltpu.VMEM((CH, D), jnp.float32),
        pltpu.VMEM_SHARED((NP, D), jnp.float32),
        pltpu.SemaphoreType.DMA,
        pltpu.SemaphoreType.DMA,
        pltpu.SemaphoreType.DMA,
        pltpu.SemaphoreType.DMA,
    ],
)
def _sc_count(dstf_hbm, zrow_hbm, cnt_hbm, dst_v, ones_v, cnt_sh,
              s0, s1, s2, s3):
    ssems = (s0, s1, s2, s3)
    c = lax.axis_index("c")
    s = lax.axis_index("s")
    w = c * NS + s
    pltpu.sync_copy(dstf_hbm.at[w], dst_v)
    pltpu.sync_copy(zrow_hbm.at[pl.ds(s * RPT, RPT)],
                    cnt_sh.at[pl.ds(s * RPT, RPT)])
    one16 = jnp.full((L,), 1.0, jnp.float32)

    def fill(t, carry):
        ones_v[t // 8, pl.ds((t % 8) * L, L)] = one16
        return carry

    lax.fori_loop(0, CH * 8, fill, 0)
    plsc.subcore_barrier()

    def scatter_desc(j, b):
        return pltpu.make_async_copy(ones_v,
                                     cnt_sh.at[dst_v.at[pl.ds(j * CH, CH)]],
                                     ssems[b])

    def batch(t, carry):
        j0 = t * 4
        for b in range(4):
            scatter_desc(j0 + b, b).start(add=True)
        for b in range(4):
            scatter_desc(j0 + b, b).wait()
        return carry

    lax.fori_loop(0, NCH // 4, batch, 0)
    for j in range(NCH - NCH % 4, NCH):
        pltpu.sync_copy(ones_v, cnt_sh.at[dst_v.at[pl.ds(j * CH, CH)]],
                        add=True)
    plsc.subcore_barrier()
    pltpu.sync_copy(cnt_sh.at[pl.ds(s * RPT, RPT)],
                    cnt_hbm.at[c, pl.ds(s * RPT, RPT)])


RB = 1024
GB = RB // D


def _lin_body(x_ref, wl_ref, wr_ref, b_ref, y_ref, z_ref):
    xv = x_ref[...]
    y_ref[...] = lax.dot_general(xv, wl_ref[...], _DN, precision=_PREC)
    z_ref[...] = lax.dot_general(xv, wr_ref[...], _DN, precision=_PREC) + b_ref[...]


def _lin(x, wl, wr, b):
    return pl.pallas_call(
        _lin_body,
        grid=(pl.cdiv(N, RB),),
        in_specs=[pl.BlockSpec((RB, D), lambda i: (i, 0)),
                  pl.BlockSpec((D, D), lambda i: (0, 0)),
                  pl.BlockSpec((D, D), lambda i: (0, 0)),
                  pl.BlockSpec((1, D), lambda i: (0, 0))],
        out_specs=[pl.BlockSpec((RB, D), lambda i: (i, 0)),
                   pl.BlockSpec((RB, D), lambda i: (i, 0))],
        out_shape=[jax.ShapeDtypeStruct((N, D), jnp.float32)] * 2,
    )(x, wl, wr, b.reshape(1, D))


def _mean(p_ref, cnt_ref):
    cnt = cnt_ref[0] + cnt_ref[1]
    return (p_ref[0] + p_ref[1]) / jnp.maximum(cnt, 1.0)


def _comb_body(p_ref, cnt_ref, z_ref, wl_ref, wr_ref, b_ref, y_ref, z2_ref):
    h = jnp.maximum(_mean(p_ref, cnt_ref) + z_ref[...], 0.0)
    y_ref[...] = lax.dot_general(h, wl_ref[...], _DN, precision=_PREC)
    z2_ref[...] = lax.dot_general(h, wr_ref[...], _DN, precision=_PREC) + b_ref[...]


def _comb(p, cnt, z, wl, wr, b):
    return pl.pallas_call(
        _comb_body,
        grid=(pl.cdiv(N, RB),),
        in_specs=[pl.BlockSpec((NC, RB, D), lambda i: (0, i, 0)),
                  pl.BlockSpec((NC, RB, D), lambda i: (0, i, 0)),
                  pl.BlockSpec((RB, D), lambda i: (i, 0)),
                  pl.BlockSpec((D, D), lambda i: (0, 0)),
                  pl.BlockSpec((D, D), lambda i: (0, 0)),
                  pl.BlockSpec((1, D), lambda i: (0, 0))],
        out_specs=[pl.BlockSpec((RB, D), lambda i: (i, 0)),
                   pl.BlockSpec((RB, D), lambda i: (i, 0))],
        out_shape=[jax.ShapeDtypeStruct((N, D), jnp.float32)] * 2,
    )(p, cnt, z, wl, wr, b.reshape(1, D))


def _final_body(p_ref, cnt_ref, z_ref, o_ref):
    o_ref[...] = _mean(p_ref, cnt_ref) + z_ref[...]


def _final(p, cnt, z):
    return pl.pallas_call(
        _final_body,
        grid=(pl.cdiv(N, RB),),
        in_specs=[pl.BlockSpec((NC, RB, D), lambda i: (0, i, 0)),
                  pl.BlockSpec((NC, RB, D), lambda i: (0, i, 0)),
                  pl.BlockSpec((RB, D), lambda i: (i, 0))],
        out_specs=pl.BlockSpec((RB, D), lambda i: (i, 0)),
        out_shape=jax.ShapeDtypeStruct((N, D), jnp.float32),
    )(p, cnt, z)


def kernel(x, edge_index, W_l1, b_l1, W_r1, W_l2, b_l2, W_r2):
    ei = edge_index.astype(jnp.int32)
    srcf = ei[0].reshape(NW, EPW)
    dstf = ei[1].reshape(NW, EPW)
    zrow = jnp.zeros((NP, D), jnp.float32)

    cnt = _sc_count(dstf, zrow)
    y1, z1 = _lin(x, W_l1, W_r1, b_l1)
    p1 = _sc_aggregate(y1, srcf, dstf, zrow)
    y2, z2 = _comb(p1, cnt, z1, W_l2, W_r2, b_l2)
    p2 = _sc_aggregate(y2, srcf, dstf, zrow)
    return _final(p2, cnt, z2)
